# baseline (speedup 1.0000x reference)
import functools

import jax
import jax.numpy as jnp
from jax import lax
from jax.experimental import pallas as pl
from jax.experimental.pallas import tpu as pltpu
from jax.experimental.pallas import tpu_sc as plsc

N = 10000
E = 320000
D = 128
H2 = 64
NC_OUT = 10
EPS = 1e-5
BN_INV = 1.0 / (1.0 + EPS) ** 0.5

NCORES = 2
NSUB = 16
NW = NCORES * NSUB
CH = 112
NCH = 90
EPAD = NW * NCH * CH
NPAD = 10048
RPT = NPAD // NSUB


def _fill_2d(ref, rows, cols, val):
    v = jnp.full((16,), val, jnp.float32)
    nc = cols // 16

    def body(i, _):
        r = i // nc
        c = i % nc
        ref[r, pl.ds(c * 16, 16)] = v
        return 0

    lax.fori_loop(0, rows * nc, body, 0)


@functools.lru_cache(maxsize=None)
def _make_sc_agg(width):
    mesh = plsc.VectorSubcoreMesh(core_axis_name="c", subcore_axis_name="s")

    def body(tbl_hbm, src_hbm, dst_hbm, acc_out,
             srcv, dstv, rows_a, rows_b, accs, ga, gb, sa, sb):
        cid = lax.axis_index("c")
        sid = lax.axis_index("s")
        wid = cid * NSUB + sid
        base = sid * RPT

        _fill_2d(rows_a, CH, width, 0.0)
        _fill_2d(rows_b, CH, width, 0.0)
        for k in range(RPT // CH):
            pltpu.sync_copy(rows_a, accs.at[pl.ds(base + k * CH, CH)])
        tail = RPT % CH
        if tail:
            pltpu.sync_copy(rows_a.at[pl.ds(0, tail)],
                            accs.at[pl.ds(base + RPT - tail, tail)])
        plsc.subcore_barrier()

        pltpu.sync_copy(src_hbm.at[wid], srcv.at[pl.ds(0, NCH)])
        zi = jnp.zeros((16,), jnp.int32)
        for k in range(CH // 16):
            srcv[NCH, pl.ds(k * 16, 16)] = zi
        pltpu.sync_copy(dst_hbm.at[wid], dstv)

        def gather(c, buf, sem):
            return pltpu.async_copy(tbl_hbm.at[srcv.at[c]], buf, sem)

        def scatter(c, buf, sem):
            return pltpu.async_copy(buf, accs.at[dstv.at[c]], sem, add=True)

        def wait_gather(buf, sem):
            pltpu.make_async_copy(tbl_hbm.at[srcv.at[0]], buf, sem).wait()

        def wait_scatter(buf, sem):
            pltpu.make_async_copy(buf, accs.at[dstv.at[0]], sem).wait()

        scatter(0, rows_b, sb)
        gather(0, rows_a, ga)

        def pair(i, _):
            c0 = 2 * i
            wait_scatter(rows_b, sb)
            gather(c0 + 1, rows_b, gb)
            wait_gather(rows_a, ga)
            scatter(c0, rows_a, sa)
            wait_scatter(rows_a, sa)
            gather(c0 + 2, rows_a, ga)
            wait_gather(rows_b, gb)
            scatter(c0 + 1, rows_b, sb)
            return 0

        lax.fori_loop(0, NCH // 2, pair, 0)
        wait_scatter(rows_b, sb)
        wait_gather(rows_a, ga)
        plsc.subcore_barrier()

        pltpu.sync_copy(accs.at[pl.ds(base, RPT)], acc_out.at[cid, pl.ds(base, RPT)])

    return pl.kernel(
        body,
        out_type=jax.ShapeDtypeStruct((NCORES, NPAD, width), jnp.float32),
        mesh=mesh,
        compiler_params=pltpu.CompilerParams(use_tc_tiling_on_sc=False),
        scratch_types=[
            pltpu.VMEM((NCH + 1, CH), jnp.int32),
            pltpu.VMEM((NCH, CH), jnp.int32),
            pltpu.VMEM((CH, width), jnp.float32),
            pltpu.VMEM((CH, width), jnp.float32),
            pltpu.VMEM_SHARED((NPAD, width), jnp.float32),
            pltpu.SemaphoreType.DMA,
            pltpu.SemaphoreType.DMA,
            pltpu.SemaphoreType.DMA,
            pltpu.SemaphoreType.DMA,
        ])


@functools.lru_cache(maxsize=None)
def _make_sc_cnt():
    mesh = plsc.VectorSubcoreMesh(core_axis_name="c", subcore_axis_name="s")

    def body(dst_hbm, cnt_out, dstv, hist):
        cid = lax.axis_index("c")
        sid = lax.axis_index("s")
        wid = cid * NSUB + sid

        z = jnp.zeros((16,), jnp.float32)

        def zb(i, _):
            hist[pl.ds(i * 16, 16)] = z
            return 0

        lax.fori_loop(0, NPAD // 16, zb, 0)

        pltpu.sync_copy(dst_hbm.at[wid], dstv)
        ones = jnp.ones((16,), jnp.float32)
        ng = CH // 16

        def g(i, _):
            ids = dstv[i // ng, pl.ds((i % ng) * 16, 16)]
            plsc.addupdate_scatter(hist, [ids], ones)
            return 0

        lax.fori_loop(0, NCH * ng, g, 0)
        pltpu.sync_copy(hist, cnt_out.at[wid])

    return pl.kernel(
        body,
        out_type=jax.ShapeDtypeStruct((NW, NPAD), jnp.float32),
        mesh=mesh,
        compiler_params=pltpu.CompilerParams(use_tc_tiling_on_sc=False,
                                             needs_layout_passes=False),
        scratch_types=[
            pltpu.VMEM((NCH, CH), jnp.int32),
            pltpu.VMEM((NPAD,), jnp.float32),
        ])


def _sc_agg_d(tbl, src3, dst3):
    return (_make_sc_agg(D)(tbl, src3, dst3), _make_sc_cnt()(dst3))


def _sc_agg_h2(tbl, src3, dst3):
    return (_make_sc_agg(H2)(tbl, src3, dst3),)


R = 1000


def _tc1_body(x_ref, p_ref, c_ref, w1l_ref, w1r_ref, b1l_ref, g1_ref, be1_ref,
              w2l_ref, w2r_ref, o1_ref, o2_ref):
    p = p_ref[0] + p_ref[1]
    cnt = jnp.maximum(jnp.sum(c_ref[...], axis=1), 1.0)[:, None]
    mean = p / cnt
    h = (jnp.dot(mean, w1l_ref[...], preferred_element_type=jnp.float32)
         + b1l_ref[...]
         + jnp.dot(x_ref[...], w1r_ref[...], preferred_element_type=jnp.float32))
    h = h * (BN_INV * g1_ref[...]) + be1_ref[...]
    h = jnp.maximum(h, 0.0)
    o1_ref[...] = jnp.dot(h, w2l_ref[...], preferred_element_type=jnp.float32)
    o2_ref[...] = jnp.dot(h, w2r_ref[...], preferred_element_type=jnp.float32)


def _tc1(x, p, c, w1l, w1r, b1l, g1, be1, w2l, w2r):
    return pl.pallas_call(
        _tc1_body,
        grid=(N // R,),
        in_specs=[
            pl.BlockSpec((R, D), lambda i: (i, 0)),
            pl.BlockSpec((NCORES, R, D), lambda i: (0, i, 0)),
            pl.BlockSpec((R, NW), lambda i: (i, 0)),
            pl.BlockSpec((D, D), lambda i: (0, 0)),
            pl.BlockSpec((D, D), lambda i: (0, 0)),
            pl.BlockSpec((1, D), lambda i: (0, 0)),
            pl.BlockSpec((1, D), lambda i: (0, 0)),
            pl.BlockSpec((1, D), lambda i: (0, 0)),
            pl.BlockSpec((D, H2), lambda i: (0, 0)),
            pl.BlockSpec((D, H2), lambda i: (0, 0)),
        ],
        out_specs=[
            pl.BlockSpec((R, H2), lambda i: (i, 0)),
            pl.BlockSpec((R, H2), lambda i: (i, 0)),
        ],
        out_shape=[
            jax.ShapeDtypeStruct((N, H2), jnp.float32),
            jax.ShapeDtypeStruct((N, H2), jnp.float32),
        ],
    )(x, p, c, w1l, w1r, b1l, g1, be1, w2l, w2r)


def _tc2_body(q_ref, c_ref, hr_ref, b2l_ref, g2_ref, be2_ref, wh_ref, bh_ref,
              o_ref):
    q = q_ref[0] + q_ref[1]
    cnt = jnp.maximum(jnp.sum(c_ref[...], axis=1), 1.0)[:, None]
    pre = q / cnt + b2l_ref[...] + hr_ref[...]
    h = jnp.maximum(pre * (BN_INV * g2_ref[...]) + be2_ref[...], 0.0)
    o_ref[...] = (jnp.dot(h, wh_ref[...], preferred_element_type=jnp.float32)
                  + bh_ref[...])


def _tc2(q, c, hr, b2l, g2, be2, wh, bh):
    return pl.pallas_call(
        _tc2_body,
        grid=(N // R,),
        in_specs=[
            pl.BlockSpec((NCORES, R, H2), lambda i: (0, i, 0)),
            pl.BlockSpec((R, NW), lambda i: (i, 0)),
            pl.BlockSpec((R, H2), lambda i: (i, 0)),
            pl.BlockSpec((1, H2), lambda i: (0, 0)),
            pl.BlockSpec((1, H2), lambda i: (0, 0)),
            pl.BlockSpec((1, H2), lambda i: (0, 0)),
            pl.BlockSpec((H2, NC_OUT), lambda i: (0, 0)),
            pl.BlockSpec((1, NC_OUT), lambda i: (0, 0)),
        ],
        out_specs=pl.BlockSpec((R, NC_OUT), lambda i: (i, 0)),
        out_shape=jax.ShapeDtypeStruct((N, NC_OUT), jnp.float32),
    )(q, c, hr, b2l, g2, be2, wh, bh)


def kernel(x, ei, W1l, b1l, W1r, g1, be1, W2l, b2l, W2r, g2, be2, Wh, bh):
    src = ei[0].astype(jnp.int32)
    dst = ei[1].astype(jnp.int32)
    src3 = jnp.concatenate(
        [src, jnp.zeros((EPAD - E,), jnp.int32)]).reshape(NW, NCH, CH)
    dst3 = jnp.concatenate(
        [dst, jnp.full((EPAD - E,), N, jnp.int32)]).reshape(NW, NCH, CH)

    p, c = _sc_agg_d(x, src3, dst3)
    c = c.T
    h2l, h2r = _tc1(x, p, c, W1l, W1r, b1l.reshape(1, D), g1.reshape(1, D),
                    be1.reshape(1, D), W2l, W2r)
    (q,) = _sc_agg_h2(h2l, src3, dst3)
    return _tc2(q, c, h2r, b2l.reshape(1, H2), g2.reshape(1, H2),
                be2.reshape(1, H2), Wh, bh.reshape(1, NC_OUT))


# speedup vs baseline: 1.0189x; 1.0189x over previous
import functools

import jax
import jax.numpy as jnp
from jax import lax
from jax.experimental import pallas as pl
from jax.experimental.pallas import tpu as pltpu
from jax.experimental.pallas import tpu_sc as plsc

N = 10000
E = 320000
D = 128
H2 = 64
NC_OUT = 10
EPS = 1e-5
BN_INV = 1.0 / (1.0 + EPS) ** 0.5

NCORES = 2
NSUB = 16
NW = NCORES * NSUB
CH = 112
NCH = 90
EPAD = NW * NCH * CH
NPAD = 10240
RPT = NPAD // NSUB


def _fill_2d(ref, rows, cols, val):
    v = jnp.full((16,), val, jnp.float32)
    nc = cols // 16

    def body(i, _):
        r = i // nc
        c = i % nc
        ref[r, pl.ds(c * 16, 16)] = v
        return 0

    lax.fori_loop(0, rows * nc, body, 0)


@functools.lru_cache(maxsize=None)
def _make_sc_agg(width):
    mesh = plsc.VectorSubcoreMesh(core_axis_name="c", subcore_axis_name="s")

    def body(tbl_hbm, src_hbm, dst_hbm, acc_out,
             srcv, dstv, rows_a, rows_b, accs, ga, gb, sa, sb):
        cid = lax.axis_index("c")
        sid = lax.axis_index("s")
        wid = cid * NSUB + sid
        base = sid * RPT

        _fill_2d(rows_a, CH, width, 0.0)
        _fill_2d(rows_b, CH, width, 0.0)
        for k in range(RPT // CH):
            pltpu.sync_copy(rows_a, accs.at[pl.ds(base + k * CH, CH)])
        tail = RPT % CH
        if tail:
            pltpu.sync_copy(rows_a.at[pl.ds(0, tail)],
                            accs.at[pl.ds(base + RPT - tail, tail)])
        plsc.subcore_barrier()

        pltpu.sync_copy(src_hbm.at[wid], srcv.at[pl.ds(0, NCH)])
        zi = jnp.zeros((16,), jnp.int32)
        for k in range(CH // 16):
            srcv[NCH, pl.ds(k * 16, 16)] = zi
        pltpu.sync_copy(dst_hbm.at[wid], dstv)

        def gather(c, buf, sem):
            return pltpu.async_copy(tbl_hbm.at[srcv.at[c]], buf, sem)

        def scatter(c, buf, sem):
            return pltpu.async_copy(buf, accs.at[dstv.at[c]], sem, add=True)

        def wait_gather(buf, sem):
            pltpu.make_async_copy(tbl_hbm.at[srcv.at[0]], buf, sem).wait()

        def wait_scatter(buf, sem):
            pltpu.make_async_copy(buf, accs.at[dstv.at[0]], sem).wait()

        scatter(0, rows_b, sb)
        gather(0, rows_a, ga)

        def pair(i, _):
            c0 = 2 * i
            wait_scatter(rows_b, sb)
            gather(c0 + 1, rows_b, gb)
            wait_gather(rows_a, ga)
            scatter(c0, rows_a, sa)
            wait_scatter(rows_a, sa)
            gather(c0 + 2, rows_a, ga)
            wait_gather(rows_b, gb)
            scatter(c0 + 1, rows_b, sb)
            return 0

        lax.fori_loop(0, NCH // 2, pair, 0)
        wait_scatter(rows_b, sb)
        wait_gather(rows_a, ga)
        plsc.subcore_barrier()

        pltpu.sync_copy(accs.at[pl.ds(base, RPT)], acc_out.at[cid, pl.ds(base, RPT)])

    return pl.kernel(
        body,
        out_type=jax.ShapeDtypeStruct((NCORES, NPAD, width), jnp.float32),
        mesh=mesh,
        compiler_params=pltpu.CompilerParams(use_tc_tiling_on_sc=False),
        scratch_types=[
            pltpu.VMEM((NCH + 1, CH), jnp.int32),
            pltpu.VMEM((NCH, CH), jnp.int32),
            pltpu.VMEM((CH, width), jnp.float32),
            pltpu.VMEM((CH, width), jnp.float32),
            pltpu.VMEM_SHARED((NPAD, width), jnp.float32),
            pltpu.SemaphoreType.DMA,
            pltpu.SemaphoreType.DMA,
            pltpu.SemaphoreType.DMA,
            pltpu.SemaphoreType.DMA,
        ])


@functools.lru_cache(maxsize=None)
def _make_sc_cnt():
    mesh = plsc.VectorSubcoreMesh(core_axis_name="c", subcore_axis_name="s")

    def body(dst_hbm, cnt_out, dstv, hist):
        cid = lax.axis_index("c")
        sid = lax.axis_index("s")
        wid = cid * NSUB + sid

        z = jnp.zeros((16,), jnp.float32)

        def zb(i, _):
            hist[pl.ds(i * 16, 16)] = z
            return 0

        lax.fori_loop(0, NPAD // 16, zb, 0)

        pltpu.sync_copy(dst_hbm.at[wid], dstv)
        ones = jnp.ones((16,), jnp.float32)
        ng = CH // 16

        def g(i, _):
            ids = dstv[i // ng, pl.ds((i % ng) * 16, 16)]
            plsc.addupdate_scatter(hist, [ids], ones)
            return 0

        lax.fori_loop(0, NCH * ng, g, 0)
        pltpu.sync_copy(hist, cnt_out.at[wid])

    return pl.kernel(
        body,
        out_type=jax.ShapeDtypeStruct((NW, NPAD), jnp.float32),
        mesh=mesh,
        compiler_params=pltpu.CompilerParams(use_tc_tiling_on_sc=False,
                                             needs_layout_passes=False),
        scratch_types=[
            pltpu.VMEM((NCH, CH), jnp.int32),
            pltpu.VMEM((NPAD,), jnp.float32),
        ])


def _sc_agg_d(tbl, src3, dst3):
    return (_make_sc_agg(D)(tbl, src3, dst3), _make_sc_cnt()(dst3))


def _sc_agg_h2(tbl, src3, dst3):
    return (_make_sc_agg(H2)(tbl, src3, dst3),)


R = 1000


def _tc1_body(x_ref, p_ref, c_ref, w1l_ref, w1r_ref, b1l_ref, g1_ref, be1_ref,
              w2l_ref, w2r_ref, o1_ref, o2_ref):
    p = p_ref[0] + p_ref[1]
    cnt = jnp.maximum(jnp.sum(c_ref[...], axis=1), 1.0)[:, None]
    mean = p / cnt
    h = (jnp.dot(mean, w1l_ref[...], preferred_element_type=jnp.float32)
         + b1l_ref[...]
         + jnp.dot(x_ref[...], w1r_ref[...], preferred_element_type=jnp.float32))
    h = h * (BN_INV * g1_ref[...]) + be1_ref[...]
    h = jnp.maximum(h, 0.0)
    o1_ref[...] = jnp.dot(h, w2l_ref[...], preferred_element_type=jnp.float32)
    o2_ref[...] = jnp.dot(h, w2r_ref[...], preferred_element_type=jnp.float32)


def _tc1(x, p, c, w1l, w1r, b1l, g1, be1, w2l, w2r):
    return pl.pallas_call(
        _tc1_body,
        grid=(N // R,),
        in_specs=[
            pl.BlockSpec((R, D), lambda i: (i, 0)),
            pl.BlockSpec((NCORES, R, D), lambda i: (0, i, 0)),
            pl.BlockSpec((R, NW), lambda i: (i, 0)),
            pl.BlockSpec((D, D), lambda i: (0, 0)),
            pl.BlockSpec((D, D), lambda i: (0, 0)),
            pl.BlockSpec((1, D), lambda i: (0, 0)),
            pl.BlockSpec((1, D), lambda i: (0, 0)),
            pl.BlockSpec((1, D), lambda i: (0, 0)),
            pl.BlockSpec((D, H2), lambda i: (0, 0)),
            pl.BlockSpec((D, H2), lambda i: (0, 0)),
        ],
        out_specs=[
            pl.BlockSpec((R, H2), lambda i: (i, 0)),
            pl.BlockSpec((R, H2), lambda i: (i, 0)),
        ],
        out_shape=[
            jax.ShapeDtypeStruct((N, H2), jnp.float32),
            jax.ShapeDtypeStruct((N, H2), jnp.float32),
        ],
    )(x, p, c, w1l, w1r, b1l, g1, be1, w2l, w2r)


def _tc2_body(q_ref, c_ref, hr_ref, b2l_ref, g2_ref, be2_ref, wh_ref, bh_ref,
              o_ref):
    q = q_ref[0] + q_ref[1]
    cnt = jnp.maximum(jnp.sum(c_ref[...], axis=1), 1.0)[:, None]
    pre = q / cnt + b2l_ref[...] + hr_ref[...]
    h = jnp.maximum(pre * (BN_INV * g2_ref[...]) + be2_ref[...], 0.0)
    o_ref[...] = (jnp.dot(h, wh_ref[...], preferred_element_type=jnp.float32)
                  + bh_ref[...])


def _tc2(q, c, hr, b2l, g2, be2, wh, bh):
    return pl.pallas_call(
        _tc2_body,
        grid=(N // R,),
        in_specs=[
            pl.BlockSpec((NCORES, R, H2), lambda i: (0, i, 0)),
            pl.BlockSpec((R, NW), lambda i: (i, 0)),
            pl.BlockSpec((R, H2), lambda i: (i, 0)),
            pl.BlockSpec((1, H2), lambda i: (0, 0)),
            pl.BlockSpec((1, H2), lambda i: (0, 0)),
            pl.BlockSpec((1, H2), lambda i: (0, 0)),
            pl.BlockSpec((H2, NC_OUT), lambda i: (0, 0)),
            pl.BlockSpec((1, NC_OUT), lambda i: (0, 0)),
        ],
        out_specs=pl.BlockSpec((R, NC_OUT), lambda i: (i, 0)),
        out_shape=jax.ShapeDtypeStruct((N, NC_OUT), jnp.float32),
    )(q, c, hr, b2l, g2, be2, wh, bh)


def kernel(x, ei, W1l, b1l, W1r, g1, be1, W2l, b2l, W2r, g2, be2, Wh, bh):
    src = ei[0].astype(jnp.int32)
    dst = ei[1].astype(jnp.int32)
    src3 = jnp.concatenate(
        [src, jnp.zeros((EPAD - E,), jnp.int32)]).reshape(NW, NCH, CH)
    dst3 = jnp.concatenate(
        [dst, jnp.full((EPAD - E,), N, jnp.int32)]).reshape(NW, NCH, CH)

    p, c = _sc_agg_d(x, src3, dst3)
    c = c.T
    h2l, h2r = _tc1(x, p, c, W1l, W1r, b1l.reshape(1, D), g1.reshape(1, D),
                    be1.reshape(1, D), W2l, W2r)
    (q,) = _sc_agg_h2(h2l, src3, dst3)
    return _tc2(q, c, h2r, b2l.reshape(1, H2), g2.reshape(1, H2),
                be2.reshape(1, H2), Wh, bh.reshape(1, NC_OUT))


# speedup vs baseline: 1.0977x; 1.0773x over previous
import functools

import jax
import jax.numpy as jnp
from jax import lax
from jax.experimental import pallas as pl
from jax.experimental.pallas import tpu as pltpu
from jax.experimental.pallas import tpu_sc as plsc

N = 10000
E = 320000
D = 128
H2 = 64
NC_OUT = 10
EPS = 1e-5
BN_INV = 1.0 / (1.0 + EPS) ** 0.5

NCORES = 2
NSUB = 16
NW = NCORES * NSUB
CH = 80
NCH = 126
EPAD = NW * NCH * CH
NPAD = 10240
RPT = NPAD // NSUB


def _fill_2d(ref, rows, cols, val):
    v = jnp.full((16,), val, jnp.float32)
    nc = cols // 16

    def body(i, _):
        r = i // nc
        c = i % nc
        ref[r, pl.ds(c * 16, 16)] = v
        return 0

    lax.fori_loop(0, rows * nc, body, 0)


@functools.lru_cache(maxsize=None)
def _make_sc_agg(width):
    mesh = plsc.VectorSubcoreMesh(core_axis_name="c", subcore_axis_name="s")

    def body(tbl_hbm, src_hbm, dst_hbm, acc_out,
             srcv, dstv, rows_a, rows_b, accs, ga, gb, sa, sb):
        cid = lax.axis_index("c")
        sid = lax.axis_index("s")
        wid = cid * NSUB + sid
        base = sid * RPT

        _fill_2d(rows_a, CH, width, 0.0)
        _fill_2d(rows_b, CH, width, 0.0)
        for k in range(RPT // CH):
            pltpu.sync_copy(rows_a, accs.at[pl.ds(base + k * CH, CH)])
        tail = RPT % CH
        if tail:
            pltpu.sync_copy(rows_a.at[pl.ds(0, tail)],
                            accs.at[pl.ds(base + RPT - tail, tail)])
        plsc.subcore_barrier()

        pltpu.sync_copy(src_hbm.at[wid], srcv.at[pl.ds(0, NCH)])
        zi = jnp.zeros((16,), jnp.int32)
        for k in range(CH // 16):
            srcv[NCH, pl.ds(k * 16, 16)] = zi
        pltpu.sync_copy(dst_hbm.at[wid], dstv)

        def gather(c, buf, sem):
            return pltpu.async_copy(tbl_hbm.at[srcv.at[c]], buf, sem)

        def scatter(c, buf, sem):
            return pltpu.async_copy(buf, accs.at[dstv.at[c]], sem, add=True)

        def wait_gather(buf, sem):
            pltpu.make_async_copy(tbl_hbm.at[srcv.at[0]], buf, sem).wait()

        def wait_scatter(buf, sem):
            pltpu.make_async_copy(buf, accs.at[dstv.at[0]], sem).wait()

        scatter(0, rows_b, sb)
        gather(0, rows_a, ga)

        def pair(i, _):
            c0 = 2 * i
            wait_scatter(rows_b, sb)
            gather(c0 + 1, rows_b, gb)
            wait_gather(rows_a, ga)
            scatter(c0, rows_a, sa)
            wait_scatter(rows_a, sa)
            gather(c0 + 2, rows_a, ga)
            wait_gather(rows_b, gb)
            scatter(c0 + 1, rows_b, sb)
            return 0

        lax.fori_loop(0, NCH // 2, pair, 0)
        wait_scatter(rows_b, sb)
        wait_gather(rows_a, ga)
        plsc.subcore_barrier()

        pltpu.sync_copy(accs.at[pl.ds(base, RPT)], acc_out.at[cid, pl.ds(base, RPT)])

    return pl.kernel(
        body,
        out_type=jax.ShapeDtypeStruct((NCORES, NPAD, width), jnp.float32),
        mesh=mesh,
        compiler_params=pltpu.CompilerParams(use_tc_tiling_on_sc=False),
        scratch_types=[
            pltpu.VMEM((NCH + 1, CH), jnp.int32),
            pltpu.VMEM((NCH, CH), jnp.int32),
            pltpu.VMEM((CH, width), jnp.float32),
            pltpu.VMEM((CH, width), jnp.float32),
            pltpu.VMEM_SHARED((NPAD, width), jnp.float32),
            pltpu.SemaphoreType.DMA,
            pltpu.SemaphoreType.DMA,
            pltpu.SemaphoreType.DMA,
            pltpu.SemaphoreType.DMA,
        ])


@functools.lru_cache(maxsize=None)
def _make_sc_cnt():
    mesh = plsc.VectorSubcoreMesh(core_axis_name="c", subcore_axis_name="s")

    def body(dst_hbm, cnt_out, dstv, hist):
        cid = lax.axis_index("c")
        sid = lax.axis_index("s")
        wid = cid * NSUB + sid

        z = jnp.zeros((16,), jnp.float32)

        def zb(i, _):
            hist[pl.ds(i * 16, 16)] = z
            return 0

        lax.fori_loop(0, NPAD // 16, zb, 0)

        pltpu.sync_copy(dst_hbm.at[wid], dstv)
        ones = jnp.ones((16,), jnp.float32)
        ng = CH // 16

        def g(i, _):
            ids = dstv[i // ng, pl.ds((i % ng) * 16, 16)]
            plsc.addupdate_scatter(hist, [ids], ones)
            return 0

        lax.fori_loop(0, NCH * ng, g, 0)
        pltpu.sync_copy(hist, cnt_out.at[wid])

    return pl.kernel(
        body,
        out_type=jax.ShapeDtypeStruct((NW, NPAD), jnp.float32),
        mesh=mesh,
        compiler_params=pltpu.CompilerParams(use_tc_tiling_on_sc=False,
                                             needs_layout_passes=False),
        scratch_types=[
            pltpu.VMEM((NCH, CH), jnp.int32),
            pltpu.VMEM((NPAD,), jnp.float32),
        ])


def _sc_agg_d(tbl, src3, dst3):
    return (_make_sc_agg(D)(tbl, src3, dst3), _make_sc_cnt()(dst3))


def _sc_agg_h2(tbl, src3, dst3):
    return (_make_sc_agg(H2)(tbl, src3, dst3),)


R = 1000


def _tc1_body(x_ref, p_ref, c_ref, w1l_ref, w1r_ref, b1l_ref, g1_ref, be1_ref,
              w2l_ref, w2r_ref, o1_ref, o2_ref):
    p = p_ref[0] + p_ref[1]
    cnt = jnp.maximum(jnp.sum(c_ref[...], axis=1), 1.0)[:, None]
    mean = p / cnt
    h = (jnp.dot(mean, w1l_ref[...], preferred_element_type=jnp.float32)
         + b1l_ref[...]
         + jnp.dot(x_ref[...], w1r_ref[...], preferred_element_type=jnp.float32))
    h = h * (BN_INV * g1_ref[...]) + be1_ref[...]
    h = jnp.maximum(h, 0.0)
    o1_ref[...] = jnp.dot(h, w2l_ref[...], preferred_element_type=jnp.float32)
    o2_ref[...] = jnp.dot(h, w2r_ref[...], preferred_element_type=jnp.float32)


def _tc1(x, p, c, w1l, w1r, b1l, g1, be1, w2l, w2r):
    return pl.pallas_call(
        _tc1_body,
        grid=(N // R,),
        in_specs=[
            pl.BlockSpec((R, D), lambda i: (i, 0)),
            pl.BlockSpec((NCORES, R, D), lambda i: (0, i, 0)),
            pl.BlockSpec((R, NW), lambda i: (i, 0)),
            pl.BlockSpec((D, D), lambda i: (0, 0)),
            pl.BlockSpec((D, D), lambda i: (0, 0)),
            pl.BlockSpec((1, D), lambda i: (0, 0)),
            pl.BlockSpec((1, D), lambda i: (0, 0)),
            pl.BlockSpec((1, D), lambda i: (0, 0)),
            pl.BlockSpec((D, H2), lambda i: (0, 0)),
            pl.BlockSpec((D, H2), lambda i: (0, 0)),
        ],
        out_specs=[
            pl.BlockSpec((R, H2), lambda i: (i, 0)),
            pl.BlockSpec((R, H2), lambda i: (i, 0)),
        ],
        out_shape=[
            jax.ShapeDtypeStruct((N, H2), jnp.float32),
            jax.ShapeDtypeStruct((N, H2), jnp.float32),
        ],
    )(x, p, c, w1l, w1r, b1l, g1, be1, w2l, w2r)


def _tc2_body(q_ref, c_ref, hr_ref, b2l_ref, g2_ref, be2_ref, wh_ref, bh_ref,
              o_ref):
    q = q_ref[0] + q_ref[1]
    cnt = jnp.maximum(jnp.sum(c_ref[...], axis=1), 1.0)[:, None]
    pre = q / cnt + b2l_ref[...] + hr_ref[...]
    h = jnp.maximum(pre * (BN_INV * g2_ref[...]) + be2_ref[...], 0.0)
    o_ref[...] = (jnp.dot(h, wh_ref[...], preferred_element_type=jnp.float32)
                  + bh_ref[...])


def _tc2(q, c, hr, b2l, g2, be2, wh, bh):
    return pl.pallas_call(
        _tc2_body,
        grid=(N // R,),
        in_specs=[
            pl.BlockSpec((NCORES, R, H2), lambda i: (0, i, 0)),
            pl.BlockSpec((R, NW), lambda i: (i, 0)),
            pl.BlockSpec((R, H2), lambda i: (i, 0)),
            pl.BlockSpec((1, H2), lambda i: (0, 0)),
            pl.BlockSpec((1, H2), lambda i: (0, 0)),
            pl.BlockSpec((1, H2), lambda i: (0, 0)),
            pl.BlockSpec((H2, NC_OUT), lambda i: (0, 0)),
            pl.BlockSpec((1, NC_OUT), lambda i: (0, 0)),
        ],
        out_specs=pl.BlockSpec((R, NC_OUT), lambda i: (i, 0)),
        out_shape=jax.ShapeDtypeStruct((N, NC_OUT), jnp.float32),
    )(q, c, hr, b2l, g2, be2, wh, bh)


def kernel(x, ei, W1l, b1l, W1r, g1, be1, W2l, b2l, W2r, g2, be2, Wh, bh):
    src = ei[0].astype(jnp.int32)
    dst = ei[1].astype(jnp.int32)
    src3 = jnp.concatenate(
        [src, jnp.zeros((EPAD - E,), jnp.int32)]).reshape(NW, NCH, CH)
    dst3 = jnp.concatenate(
        [dst, jnp.full((EPAD - E,), N, jnp.int32)]).reshape(NW, NCH, CH)

    p, c = _sc_agg_d(x, src3, dst3)
    c = c.T
    h2l, h2r = _tc1(x, p, c, W1l, W1r, b1l.reshape(1, D), g1.reshape(1, D),
                    be1.reshape(1, D), W2l, W2r)
    (q,) = _sc_agg_h2(h2l, src3, dst3)
    return _tc2(q, c, h2r, b2l.reshape(1, H2), g2.reshape(1, H2),
                be2.reshape(1, H2), Wh, bh.reshape(1, NC_OUT))


# speedup vs baseline: 1.1011x; 1.0032x over previous
import functools

import jax
import jax.numpy as jnp
from jax import lax
from jax.experimental import pallas as pl
from jax.experimental.pallas import tpu as pltpu
from jax.experimental.pallas import tpu_sc as plsc

N = 10000
E = 320000
D = 128
H2 = 64
NC_OUT = 10
EPS = 1e-5
BN_INV = 1.0 / (1.0 + EPS) ** 0.5

NCORES = 2
NSUB = 16
NW = NCORES * NSUB
CH = 80
NCH = 126
EPAD = NW * NCH * CH
NPAD = 10240
RPT = NPAD // NSUB


def _fill_2d(ref, rows, cols, val):
    v = jnp.full((16,), val, jnp.float32)
    nc = cols // 16

    def body(i, _):
        r = i // nc
        c = i % nc
        ref[r, pl.ds(c * 16, 16)] = v
        return 0

    lax.fori_loop(0, rows * nc, body, 0)


@functools.lru_cache(maxsize=None)
def _make_sc_agg(width):
    mesh = plsc.VectorSubcoreMesh(core_axis_name="c", subcore_axis_name="s")

    def body(tbl_hbm, src_hbm, dst_hbm, acc_out,
             srcv, dstv, rows_a, rows_b, accs, ga, gb, sa, sb):
        cid = lax.axis_index("c")
        sid = lax.axis_index("s")
        wid = cid * NSUB + sid
        base = sid * RPT

        _fill_2d(rows_a, CH, width, 0.0)
        _fill_2d(rows_b, CH, width, 0.0)
        for k in range(RPT // CH):
            pltpu.sync_copy(rows_a, accs.at[pl.ds(base + k * CH, CH)])
        tail = RPT % CH
        if tail:
            pltpu.sync_copy(rows_a.at[pl.ds(0, tail)],
                            accs.at[pl.ds(base + RPT - tail, tail)])
        plsc.subcore_barrier()

        pltpu.sync_copy(src_hbm.at[wid], srcv.at[pl.ds(0, NCH)])
        zi = jnp.zeros((16,), jnp.int32)
        for k in range(CH // 16):
            srcv[NCH, pl.ds(k * 16, 16)] = zi
        pltpu.sync_copy(dst_hbm.at[wid], dstv)

        def gather(c, buf, sem):
            return pltpu.async_copy(tbl_hbm.at[srcv.at[c]], buf, sem)

        def scatter(c, buf, sem):
            return pltpu.async_copy(buf, accs.at[dstv.at[c]], sem, add=True)

        def wait_gather(buf, sem):
            pltpu.make_async_copy(tbl_hbm.at[srcv.at[0]], buf, sem).wait()

        def wait_scatter(buf, sem):
            pltpu.make_async_copy(buf, accs.at[dstv.at[0]], sem).wait()

        scatter(0, rows_b, sb)
        gather(0, rows_a, ga)

        def pair(i, _):
            c0 = 2 * i
            wait_scatter(rows_b, sb)
            gather(c0 + 1, rows_b, gb)
            wait_gather(rows_a, ga)
            scatter(c0, rows_a, sa)
            wait_scatter(rows_a, sa)
            gather(c0 + 2, rows_a, ga)
            wait_gather(rows_b, gb)
            scatter(c0 + 1, rows_b, sb)
            return 0

        lax.fori_loop(0, NCH // 2, pair, 0)
        wait_scatter(rows_b, sb)
        wait_gather(rows_a, ga)
        plsc.subcore_barrier()

        pltpu.sync_copy(accs.at[pl.ds(base, RPT)], acc_out.at[cid, pl.ds(base, RPT)])

    return pl.kernel(
        body,
        out_type=jax.ShapeDtypeStruct((NCORES, NPAD, width), jnp.float32),
        mesh=mesh,
        compiler_params=pltpu.CompilerParams(use_tc_tiling_on_sc=False),
        scratch_types=[
            pltpu.VMEM((NCH + 1, CH), jnp.int32),
            pltpu.VMEM((NCH, CH), jnp.int32),
            pltpu.VMEM((CH, width), jnp.float32),
            pltpu.VMEM((CH, width), jnp.float32),
            pltpu.VMEM_SHARED((NPAD, width), jnp.float32),
            pltpu.SemaphoreType.DMA,
            pltpu.SemaphoreType.DMA,
            pltpu.SemaphoreType.DMA,
            pltpu.SemaphoreType.DMA,
        ])


@functools.lru_cache(maxsize=None)
def _make_sc_cnt():
    mesh = plsc.VectorSubcoreMesh(core_axis_name="c", subcore_axis_name="s")

    def body(dst_hbm, cnt_out, dstv, hist):
        cid = lax.axis_index("c")
        sid = lax.axis_index("s")
        wid = cid * NSUB + sid

        z = jnp.zeros((16,), jnp.float32)

        def zb(i, _):
            hist[pl.ds(i * 16, 16)] = z
            return 0

        lax.fori_loop(0, NPAD // 16, zb, 0)

        pltpu.sync_copy(dst_hbm.at[wid], dstv)
        ones = jnp.ones((16,), jnp.float32)
        ng = CH // 16

        def g(i, _):
            ids = dstv[i // ng, pl.ds((i % ng) * 16, 16)]
            plsc.addupdate_scatter(hist, [ids], ones)
            return 0

        lax.fori_loop(0, NCH * ng, g, 0)
        pltpu.sync_copy(hist, cnt_out.at[wid])

    return pl.kernel(
        body,
        out_type=jax.ShapeDtypeStruct((NW, NPAD), jnp.float32),
        mesh=mesh,
        compiler_params=pltpu.CompilerParams(use_tc_tiling_on_sc=False,
                                             needs_layout_passes=False),
        scratch_types=[
            pltpu.VMEM((NCH, CH), jnp.int32),
            pltpu.VMEM((NPAD,), jnp.float32),
        ])


def _sc_agg_d(tbl, src3, dst3):
    return (_make_sc_agg(D)(tbl, src3, dst3), _make_sc_cnt()(dst3))


def _sc_agg_h2(tbl, src3, dst3):
    return (_make_sc_agg(H2)(tbl, src3, dst3),)


R = 1000


def _tc1_body(x_ref, p_ref, c_ref, w1l_ref, w1r_ref, b1l_ref, g1_ref, be1_ref,
              w2l_ref, w2r_ref, o1_ref, o2_ref):
    p = p_ref[0] + p_ref[1]
    cnt = jnp.maximum(jnp.sum(c_ref[...], axis=1), 1.0)[:, None]
    mean = p / cnt
    h = (jnp.dot(mean, w1l_ref[...], preferred_element_type=jnp.float32)
         + b1l_ref[...]
         + jnp.dot(x_ref[...], w1r_ref[...], preferred_element_type=jnp.float32))
    h = h * (BN_INV * g1_ref[...]) + be1_ref[...]
    h = jnp.maximum(h, 0.0)
    o1_ref[...] = jnp.dot(h, w2l_ref[...], preferred_element_type=jnp.float32)
    o2_ref[...] = jnp.dot(h, w2r_ref[...], preferred_element_type=jnp.float32)


def _tc1(x, p, c, w1l, w1r, b1l, g1, be1, w2l, w2r):
    return pl.pallas_call(
        _tc1_body,
        grid=(N // R,),
        in_specs=[
            pl.BlockSpec((R, D), lambda i: (i, 0)),
            pl.BlockSpec((NCORES, R, D), lambda i: (0, i, 0)),
            pl.BlockSpec((R, NW), lambda i: (i, 0)),
            pl.BlockSpec((D, D), lambda i: (0, 0)),
            pl.BlockSpec((D, D), lambda i: (0, 0)),
            pl.BlockSpec((1, D), lambda i: (0, 0)),
            pl.BlockSpec((1, D), lambda i: (0, 0)),
            pl.BlockSpec((1, D), lambda i: (0, 0)),
            pl.BlockSpec((D, H2), lambda i: (0, 0)),
            pl.BlockSpec((D, H2), lambda i: (0, 0)),
        ],
        out_specs=[
            pl.BlockSpec((R, H2), lambda i: (i, 0)),
            pl.BlockSpec((R, H2), lambda i: (i, 0)),
        ],
        out_shape=[
            jax.ShapeDtypeStruct((N, H2), jnp.float32),
            jax.ShapeDtypeStruct((N, H2), jnp.float32),
        ],
    )(x, p, c, w1l, w1r, b1l, g1, be1, w2l, w2r)


def _tc2_body(q_ref, c_ref, hr_ref, b2l_ref, g2_ref, be2_ref, wh_ref, bh_ref,
              o_ref):
    q = q_ref[0] + q_ref[1]
    cnt = jnp.maximum(jnp.sum(c_ref[...], axis=1), 1.0)[:, None]
    pre = q / cnt + b2l_ref[...] + hr_ref[...]
    h = jnp.maximum(pre * (BN_INV * g2_ref[...]) + be2_ref[...], 0.0)
    o_ref[...] = (jnp.dot(h, wh_ref[...], preferred_element_type=jnp.float32)
                  + bh_ref[...])


def _tc2(q, c, hr, b2l, g2, be2, wh, bh):
    return pl.pallas_call(
        _tc2_body,
        grid=(N // R,),
        in_specs=[
            pl.BlockSpec((NCORES, R, H2), lambda i: (0, i, 0)),
            pl.BlockSpec((R, NW), lambda i: (i, 0)),
            pl.BlockSpec((R, H2), lambda i: (i, 0)),
            pl.BlockSpec((1, H2), lambda i: (0, 0)),
            pl.BlockSpec((1, H2), lambda i: (0, 0)),
            pl.BlockSpec((1, H2), lambda i: (0, 0)),
            pl.BlockSpec((H2, NC_OUT), lambda i: (0, 0)),
            pl.BlockSpec((1, NC_OUT), lambda i: (0, 0)),
        ],
        out_specs=pl.BlockSpec((R, NC_OUT), lambda i: (i, 0)),
        out_shape=jax.ShapeDtypeStruct((N, NC_OUT), jnp.float32),
    )(q, c, hr, b2l, g2, be2, wh, bh)


def kernel(x, ei, W1l, b1l, W1r, g1, be1, W2l, b2l, W2r, g2, be2, Wh, bh):
    src = ei[0].astype(jnp.int32)
    dst = ei[1].astype(jnp.int32)
    pad_dst = N + jnp.arange(EPAD - E, dtype=jnp.int32) % (NPAD - N)
    src3 = jnp.concatenate(
        [src, jnp.zeros((EPAD - E,), jnp.int32)]).reshape(NW, NCH, CH)
    dst3 = jnp.concatenate([dst, pad_dst]).reshape(NW, NCH, CH)

    p, c = _sc_agg_d(x, src3, dst3)
    c = c.T
    h2l, h2r = _tc1(x, p, c, W1l, W1r, b1l.reshape(1, D), g1.reshape(1, D),
                    be1.reshape(1, D), W2l, W2r)
    (q,) = _sc_agg_h2(h2l, src3, dst3)
    return _tc2(q, c, h2r, b2l.reshape(1, H2), g2.reshape(1, H2),
                be2.reshape(1, H2), Wh, bh.reshape(1, NC_OUT))


# speedup vs baseline: 1.5076x; 1.3692x over previous
import functools

import jax
import jax.numpy as jnp
from jax import lax
from jax.experimental import pallas as pl
from jax.experimental.pallas import tpu as pltpu
from jax.experimental.pallas import tpu_sc as plsc

N = 10000
E = 320000
D = 128
H2 = 64
NC_OUT = 10
EPS = 1e-5
BN_INV = 1.0 / (1.0 + EPS) ** 0.5

NCORES = 2
NSUB = 16
NW = NCORES * NSUB
CH = 80
NCH0 = 126
NCH1 = 124
NCH = NCH0
NPAD = 10240
RPT = NPAD // NSUB


def _fill_2d(ref, rows, cols, val):
    v = jnp.full((16,), val, jnp.float32)
    nc = cols // 16

    def body(i, _):
        r = i // nc
        c = i % nc
        ref[r, pl.ds(c * 16, 16)] = v
        return 0

    lax.fori_loop(0, rows * nc, body, 0)


@functools.lru_cache(maxsize=None)
def _make_sc_agg(width, pairs0=NCH0 // 2, pairs1=NCH1 // 2):
    mesh = plsc.VectorSubcoreMesh(core_axis_name="c", subcore_axis_name="s")

    def body(tbl_hbm, src_hbm, dst_hbm, acc_out,
             srcv, dstv, rows_a, rows_b, accs, ga, gb, sa, sb):
        cid = lax.axis_index("c")
        sid = lax.axis_index("s")
        wid = cid * NSUB + sid
        base = sid * RPT

        _fill_2d(rows_a, CH, width, 0.0)
        _fill_2d(rows_b, CH, width, 0.0)
        for k in range(RPT // CH):
            pltpu.sync_copy(rows_a, accs.at[pl.ds(base + k * CH, CH)])
        tail = RPT % CH
        if tail:
            pltpu.sync_copy(rows_a.at[pl.ds(0, tail)],
                            accs.at[pl.ds(base + RPT - tail, tail)])
        plsc.subcore_barrier()

        pltpu.sync_copy(src_hbm.at[wid], srcv.at[pl.ds(0, NCH)])
        zi = jnp.zeros((16,), jnp.int32)
        for k in range(CH // 16):
            srcv[NCH, pl.ds(k * 16, 16)] = zi
        pltpu.sync_copy(dst_hbm.at[wid], dstv)

        def gather(c, buf, sem):
            return pltpu.async_copy(tbl_hbm.at[srcv.at[c]], buf, sem)

        def scatter(c, buf, sem):
            return pltpu.async_copy(buf, accs.at[dstv.at[c]], sem, add=True)

        def wait_gather(buf, sem):
            pltpu.make_async_copy(tbl_hbm.at[srcv.at[0]], buf, sem).wait()

        def wait_scatter(buf, sem):
            pltpu.make_async_copy(buf, accs.at[dstv.at[0]], sem).wait()

        scatter(0, rows_b, sb)
        gather(0, rows_a, ga)

        def pair(i, _):
            c0 = 2 * i
            wait_scatter(rows_b, sb)
            gather(c0 + 1, rows_b, gb)
            wait_gather(rows_a, ga)
            scatter(c0, rows_a, sa)
            wait_scatter(rows_a, sa)
            gather(c0 + 2, rows_a, ga)
            wait_gather(rows_b, gb)
            scatter(c0 + 1, rows_b, sb)
            return 0

        npairs = jnp.where(cid == 0, pairs0, pairs1)
        lax.fori_loop(0, npairs, pair, 0)
        wait_scatter(rows_b, sb)
        wait_gather(rows_a, ga)
        plsc.subcore_barrier()

        pltpu.sync_copy(accs.at[pl.ds(base, RPT)], acc_out.at[cid, pl.ds(base, RPT)])

    return pl.kernel(
        body,
        out_type=jax.ShapeDtypeStruct((NCORES, NPAD, width), jnp.float32),
        mesh=mesh,
        compiler_params=pltpu.CompilerParams(use_tc_tiling_on_sc=False),
        scratch_types=[
            pltpu.VMEM((NCH + 1, CH), jnp.int32),
            pltpu.VMEM((NCH, CH), jnp.int32),
            pltpu.VMEM((CH, width), jnp.float32),
            pltpu.VMEM((CH, width), jnp.float32),
            pltpu.VMEM_SHARED((NPAD, width), jnp.float32),
            pltpu.SemaphoreType.DMA,
            pltpu.SemaphoreType.DMA,
            pltpu.SemaphoreType.DMA,
            pltpu.SemaphoreType.DMA,
        ])


@functools.lru_cache(maxsize=None)
def _make_sc_cnt():
    mesh = plsc.VectorSubcoreMesh(core_axis_name="c", subcore_axis_name="s")

    def body(dst_hbm, cnt_out, dstv, hist):
        cid = lax.axis_index("c")
        sid = lax.axis_index("s")
        wid = cid * NSUB + sid

        z = jnp.zeros((16,), jnp.float32)

        def zb(i, _):
            hist[pl.ds(i * 16, 16)] = z
            return 0

        lax.fori_loop(0, NPAD // 16, zb, 0)

        pltpu.sync_copy(dst_hbm.at[wid], dstv)
        ones = jnp.ones((16,), jnp.float32)
        ng = CH // 16

        def g(i, _):
            ids = dstv[i // ng, pl.ds((i % ng) * 16, 16)]
            plsc.addupdate_scatter(hist, [ids], ones)
            return 0

        lax.fori_loop(0, NCH * ng, g, 0)
        pltpu.sync_copy(hist, cnt_out.at[wid])

    return pl.kernel(
        body,
        out_type=jax.ShapeDtypeStruct((NW, NPAD), jnp.float32),
        mesh=mesh,
        compiler_params=pltpu.CompilerParams(use_tc_tiling_on_sc=False,
                                             needs_layout_passes=False),
        scratch_types=[
            pltpu.VMEM((NCH, CH), jnp.int32),
            pltpu.VMEM((NPAD,), jnp.float32),
        ])


def _sc_agg_d(tbl, src3, dst3):
    return (_make_sc_agg(D)(tbl, src3, dst3), _make_sc_cnt()(dst3))


def _sc_agg_h2(tbl, src3, dst3):
    return (_make_sc_agg(H2)(tbl, src3, dst3),)


R = 1000


def _tc1_body(x_ref, p_ref, c_ref, w1l_ref, w1r_ref, b1l_ref, g1_ref, be1_ref,
              w2l_ref, w2r_ref, o1_ref, o2_ref):
    p = p_ref[0] + p_ref[1]
    cnt = jnp.maximum(jnp.sum(c_ref[...], axis=1), 1.0)[:, None]
    mean = p / cnt
    h = (jnp.dot(mean, w1l_ref[...], preferred_element_type=jnp.float32)
         + b1l_ref[...]
         + jnp.dot(x_ref[...], w1r_ref[...], preferred_element_type=jnp.float32))
    h = h * (BN_INV * g1_ref[...]) + be1_ref[...]
    h = jnp.maximum(h, 0.0)
    o1_ref[...] = jnp.dot(h, w2l_ref[...], preferred_element_type=jnp.float32)
    o2_ref[...] = jnp.dot(h, w2r_ref[...], preferred_element_type=jnp.float32)


def _tc1(x, p, c, w1l, w1r, b1l, g1, be1, w2l, w2r):
    return pl.pallas_call(
        _tc1_body,
        grid=(N // R,),
        in_specs=[
            pl.BlockSpec((R, D), lambda i: (i, 0)),
            pl.BlockSpec((NCORES, R, D), lambda i: (0, i, 0)),
            pl.BlockSpec((R, NW), lambda i: (i, 0)),
            pl.BlockSpec((D, D), lambda i: (0, 0)),
            pl.BlockSpec((D, D), lambda i: (0, 0)),
            pl.BlockSpec((1, D), lambda i: (0, 0)),
            pl.BlockSpec((1, D), lambda i: (0, 0)),
            pl.BlockSpec((1, D), lambda i: (0, 0)),
            pl.BlockSpec((D, H2), lambda i: (0, 0)),
            pl.BlockSpec((D, H2), lambda i: (0, 0)),
        ],
        out_specs=[
            pl.BlockSpec((R, H2), lambda i: (i, 0)),
            pl.BlockSpec((R, H2), lambda i: (i, 0)),
        ],
        out_shape=[
            jax.ShapeDtypeStruct((N, H2), jnp.float32),
            jax.ShapeDtypeStruct((N, H2), jnp.float32),
        ],
    )(x, p, c, w1l, w1r, b1l, g1, be1, w2l, w2r)


def _tc2_body(q_ref, c_ref, hr_ref, b2l_ref, g2_ref, be2_ref, wh_ref, bh_ref,
              o_ref):
    q = q_ref[0] + q_ref[1]
    cnt = jnp.maximum(jnp.sum(c_ref[...], axis=1), 1.0)[:, None]
    pre = q / cnt + b2l_ref[...] + hr_ref[...]
    h = jnp.maximum(pre * (BN_INV * g2_ref[...]) + be2_ref[...], 0.0)
    o_ref[...] = (jnp.dot(h, wh_ref[...], preferred_element_type=jnp.float32)
                  + bh_ref[...])


def _tc2(q, c, hr, b2l, g2, be2, wh, bh):
    return pl.pallas_call(
        _tc2_body,
        grid=(N // R,),
        in_specs=[
            pl.BlockSpec((NCORES, R, H2), lambda i: (0, i, 0)),
            pl.BlockSpec((R, NW), lambda i: (i, 0)),
            pl.BlockSpec((R, H2), lambda i: (i, 0)),
            pl.BlockSpec((1, H2), lambda i: (0, 0)),
            pl.BlockSpec((1, H2), lambda i: (0, 0)),
            pl.BlockSpec((1, H2), lambda i: (0, 0)),
            pl.BlockSpec((H2, NC_OUT), lambda i: (0, 0)),
            pl.BlockSpec((1, NC_OUT), lambda i: (0, 0)),
        ],
        out_specs=pl.BlockSpec((R, NC_OUT), lambda i: (i, 0)),
        out_shape=jax.ShapeDtypeStruct((N, NC_OUT), jnp.float32),
    )(q, c, hr, b2l, g2, be2, wh, bh)


def kernel(x, ei, W1l, b1l, W1r, g1, be1, W2l, b2l, W2r, g2, be2, Wh, bh):
    src = ei[0].astype(jnp.int32)
    dst = ei[1].astype(jnp.int32)
    split = 16 * NCH0 * CH
    s0 = src[:split].reshape(16, NCH0, CH)
    d0 = dst[:split].reshape(16, NCH0, CH)
    s1 = src[split:].reshape(16, NCH1, CH)
    d1 = dst[split:].reshape(16, NCH1, CH)
    src3 = jnp.concatenate([s0, jnp.concatenate(
        [s1, jnp.zeros((16, NCH - NCH1, CH), jnp.int32)], axis=1)], axis=0)
    dst3 = jnp.concatenate([d0, jnp.concatenate(
        [d1, jnp.full((16, NCH - NCH1, CH), N, jnp.int32)], axis=1)], axis=0)

    p, c = _sc_agg_d(x, src3, dst3)
    c = c.T
    h2l, h2r = _tc1(x, p, c, W1l, W1r, b1l.reshape(1, D), g1.reshape(1, D),
                    be1.reshape(1, D), W2l, W2r)
    (q,) = _sc_agg_h2(h2l, src3, dst3)
    return _tc2(q, c, h2r, b2l.reshape(1, H2), g2.reshape(1, H2),
                be2.reshape(1, H2), Wh, bh.reshape(1, NC_OUT))
